# baseline (speedup 1.0000x reference)
import functools

import jax
import jax.numpy as jnp
from jax import lax
from jax.experimental import pallas as pl
from jax.experimental.pallas import tpu as pltpu
from jax.experimental.pallas import tpu_sc as plsc

N, E, C, K = 10000, 320000, 128, 27
EPS = 1e-5

NC, NS = 2, 16
ES = E // NS
NH = N // NC
NPADH = 5120
DUMMY = NH
CH = 128
NCH = 160
NCH2 = NCH // 2
NQ = 4
QS = 5120
ZR = NPADH // NS

_mesh = plsc.VectorSubcoreMesh(core_axis_name="c", subcore_axis_name="s",
                               num_cores=NC, num_subcores=NS)


@functools.partial(
    pl.kernel,
    out_type=pltpu.HBM((NC * NPADH, C), jnp.float32),
    mesh=_mesh,
    scratch_types=[
        pltpu.VMEM((QS,), jnp.int32),
        pltpu.VMEM((QS,), jnp.int32),
        pltpu.VMEM((QS,), jnp.int32),
        pltpu.VMEM((NCH, CH), jnp.int32),
        pltpu.VMEM((NCH, CH), jnp.int32),
        pltpu.VMEM((CH, C), jnp.float32),
        pltpu.VMEM((CH, C), jnp.float32),
        pltpu.VMEM_SHARED((NPADH, C), jnp.float32),
        pltpu.SemaphoreType.DMA,
        pltpu.SemaphoreType.DMA,
    ],
)
def _sc_gather_segsum(src_h, dst_h, off_h, xk_h, out_h,
                      src_v, off_v, dst_v, gidx, sidx,
                      rows_a, rows_b, acc, sem_a, sem_b):
    c = lax.axis_index("c")
    s = lax.axis_index("s")
    base = s * ES
    lo = c * NH

    lane = lax.iota(jnp.int32, 16)

    for q in range(NQ):
        qn = min(QS, ES - q * QS)
        pltpu.sync_copy(src_h.at[pl.ds(base + q * QS, qn)],
                        src_v.at[pl.ds(0, qn)])
        pltpu.sync_copy(off_h.at[pl.ds(base + q * QS, qn)],
                        off_v.at[pl.ds(0, qn)])
        pltpu.sync_copy(dst_h.at[pl.ds(base + q * QS, qn)],
                        dst_v.at[pl.ds(0, qn)])

        def _build(r, carry, q=q, qn=qn):
            for l in range(CH // 16):
                p0 = r * CH + l * 16
                sv = src_v[pl.ds(p0, 16)]
                ov = off_v[pl.ds(p0, 16)]
                dv = dst_v[pl.ds(p0, 16)]
                keep = (p0 + lane < qn) & (dv >= lo) & (dv < lo + NH)
                gidx[q * (QS // CH) + r, pl.ds(l * 16, 16)] = (
                    jnp.where(keep, ov * N + sv, 0))
                sidx[q * (QS // CH) + r, pl.ds(l * 16, 16)] = (
                    jnp.where(keep, dv - lo, DUMMY))
            return carry

        lax.fori_loop(0, QS // CH, _build, 0)

    zero16 = jnp.zeros((16,), jnp.float32)

    def _zrow(r, carry):
        for l in range(C // 16):
            rows_a[r, pl.ds(l * 16, 16)] = zero16
        return carry

    lax.fori_loop(0, CH, _zrow, 0)
    pltpu.sync_copy(rows_a, acc.at[pl.ds(s * ZR, CH)])
    pltpu.sync_copy(rows_a, acc.at[pl.ds(s * ZR + CH, CH)])
    pltpu.sync_copy(rows_a.at[pl.ds(0, ZR - 2 * CH)],
                    acc.at[pl.ds(s * ZR + 2 * CH, ZR - 2 * CH)])
    plsc.subcore_barrier()

    def _start(rows, sem, ci):
        pltpu.async_copy(xk_h.at[gidx.at[ci]], rows, sem)

    def _wait(rows, sem):
        pltpu.make_async_copy(xk_h.at[gidx.at[0]], rows, sem).wait()

    _start(rows_a, sem_a, 0)

    def _step(t, carry):
        c0 = 2 * t
        _start(rows_b, sem_b, c0 + 1)
        _wait(rows_a, sem_a)
        pltpu.sync_copy(rows_a, acc.at[sidx.at[c0]], add=True)

        @pl.when(t + 1 < NCH2)
        def _():
            _start(rows_a, sem_a, c0 + 2)

        _wait(rows_b, sem_b)
        pltpu.sync_copy(rows_b, acc.at[sidx.at[c0 + 1]], add=True)
        return carry

    lax.fori_loop(0, NCH2, _step, 0)

    plsc.subcore_barrier()
    pltpu.sync_copy(acc.at[pl.ds(s * ZR, ZR)],
                    out_h.at[pl.ds(c * NPADH + s * ZR, ZR)])


BN1 = 1000
NB = N // BN1
NBH = NH // BN1


def _mm1_body(x_ref, w_ref, o_ref):
    o_ref[0] = jnp.dot(x_ref[...], w_ref[0], preferred_element_type=jnp.float32)


def _einsum_xw(xin, W):
    return pl.pallas_call(
        _mm1_body,
        grid=(NB, K),
        in_specs=[pl.BlockSpec((BN1, C), lambda nb, k: (nb, 0)),
                  pl.BlockSpec((1, C, C), lambda nb, k: (k, 0, 0))],
        out_specs=pl.BlockSpec((1, BN1, C), lambda nb, k: (k, nb, 0)),
        out_shape=jax.ShapeDtypeStruct((K, N, C), jnp.float32),
    )(xin, W)


_pblk = pl.BlockSpec((1, BN1, C), lambda i, *_: (i // NBH, i % NBH, 0))


def _stats_body(p_ref, s_ref, q_ref):
    y = p_ref[0]

    @pl.when(pl.program_id(0) == 0)
    def _():
        s_ref[...] = jnp.zeros_like(s_ref)
        q_ref[...] = jnp.zeros_like(q_ref)

    s_ref[...] += jnp.sum(y, axis=0, keepdims=True)
    q_ref[...] += jnp.sum(y * y, axis=0, keepdims=True)


def _stats(p):
    return pl.pallas_call(
        _stats_body,
        grid=(NB,),
        in_specs=[_pblk],
        out_specs=[pl.BlockSpec((1, C), lambda i: (0, 0)),
                   pl.BlockSpec((1, C), lambda i: (0, 0))],
        out_shape=[jax.ShapeDtypeStruct((1, C), jnp.float32),
                   jax.ShapeDtypeStruct((1, C), jnp.float32)],
    )(p)


def _mm2_body(p_ref, s_ref, q_ref, g_ref, b_ref, w_ref, o_ref):
    mu = s_ref[0] * (1.0 / N)
    var = q_ref[0] * (1.0 / N) - mu * mu
    inv = lax.rsqrt(var + EPS) * g_ref[0]
    yn = jnp.maximum((p_ref[0] - mu) * inv + b_ref[0], 0.0)
    o_ref[0] = jnp.dot(yn, w_ref[0], preferred_element_type=jnp.float32)


def _einsum_bn_relu(p, ssum, sq, gamma, beta, W):
    vec = pl.BlockSpec((1, C), lambda nb, k: (0, 0))
    return pl.pallas_call(
        _mm2_body,
        grid=(NB, K),
        in_specs=[_pblk, vec, vec, vec, vec,
                  pl.BlockSpec((1, C, C), lambda nb, k: (k, 0, 0))],
        out_specs=pl.BlockSpec((1, BN1, C), lambda nb, k: (k, nb, 0)),
        out_shape=jax.ShapeDtypeStruct((K, N, C), jnp.float32),
    )(p, ssum, sq, gamma, beta, W)


def _fin_body(p_ref, s_ref, q_ref, g_ref, b_ref, x_ref, o_ref):
    mu = s_ref[0] * (1.0 / N)
    var = q_ref[0] * (1.0 / N) - mu * mu
    inv = lax.rsqrt(var + EPS) * g_ref[0]
    o_ref[...] = jnp.maximum((p_ref[0] - mu) * inv + b_ref[0] + x_ref[...],
                             0.0)


def _final(p, ssum, sq, gamma, beta, x):
    blk = pl.BlockSpec((BN1, C), lambda i: (i, 0))
    vec = pl.BlockSpec((1, C), lambda i: (0, 0))
    return pl.pallas_call(
        _fin_body,
        grid=(NB,),
        in_specs=[_pblk, vec, vec, vec, vec, blk],
        out_specs=blk,
        out_shape=jax.ShapeDtypeStruct((N, C), jnp.float32),
    )(p, ssum, sq, gamma, beta, x)


def kernel(x, edge_index, kernel_offset, W1, gamma1, beta1, W2, gamma2, beta2):
    src = edge_index[0]
    dst = edge_index[1]
    off = kernel_offset
    g1 = gamma1.reshape(1, C)
    b1 = beta1.reshape(1, C)
    g2 = gamma2.reshape(1, C)
    b2 = beta2.reshape(1, C)

    src_r = jax.new_ref(src, memory_space=pltpu.MemorySpace.HBM)
    dst_r = jax.new_ref(dst, memory_space=pltpu.MemorySpace.HBM)
    off_r = jax.new_ref(off, memory_space=pltpu.MemorySpace.HBM)

    xk1 = _einsum_xw(x, W1).reshape(K * N, C)
    xk1_r = jax.new_ref(xk1, memory_space=pltpu.MemorySpace.HBM)
    p1 = _sc_gather_segsum(src_r, dst_r, off_r, xk1_r).reshape(NC, NPADH, C)
    s1, q1 = _stats(p1)
    xk2 = _einsum_bn_relu(p1, s1, q1, g1, b1, W2).reshape(K * N, C)
    xk2_r = jax.new_ref(xk2, memory_space=pltpu.MemorySpace.HBM)
    p2 = _sc_gather_segsum(src_r, dst_r, off_r, xk2_r).reshape(NC, NPADH, C)
    s2, q2 = _stats(p2)
    return _final(p2, s2, q2, g2, b2, x)


# speedup vs baseline: 21.1658x; 21.1658x over previous
import functools

import jax
import jax.numpy as jnp
from jax import lax
from jax.experimental import pallas as pl
from jax.experimental.pallas import tpu as pltpu
from jax.experimental.pallas import tpu_sc as plsc

N, E, C, K = 10000, 320000, 128, 27
EPS = 1e-5

NC, NS = 2, 16
EW = E // (NC * NS)
CH = 128
NCH = 80
EWP = NCH * CH
NPH = NCH // 2
NPAD = 10240
DUMMY = N
ZR = NPAD // NS
NBUF = 2

_mesh = plsc.VectorSubcoreMesh(core_axis_name="c", subcore_axis_name="s",
                               num_cores=NC, num_subcores=NS)


@functools.partial(
    pl.kernel,
    out_type=pltpu.HBM((NC * NPAD, C), jnp.float32),
    mesh=_mesh,
    scratch_types=[
        pltpu.VMEM((NPH, CH), jnp.int32),
        pltpu.VMEM((NPH, CH), jnp.int32),
        pltpu.VMEM((CH, C), jnp.float32),
        pltpu.VMEM((CH, C), jnp.float32),
        pltpu.VMEM_SHARED((NPAD, C), jnp.float32),
        pltpu.SemaphoreType.DMA,
        pltpu.SemaphoreType.DMA,
    ],
)
def _sc_gather_segsum(gidx_h, sidx_h, xk_h, out_h,
                      gidx, sidx, rows0, rows1, acc, sem0, sem1):
    c = lax.axis_index("c")
    s = lax.axis_index("s")
    wid = c * NS + s
    rows = (rows0, rows1)
    sems = (sem0, sem1)

    zero16 = jnp.zeros((16,), jnp.float32)

    def _zrow(r, carry):
        for l in range(C // 16):
            rows0[r, pl.ds(l * 16, 16)] = zero16
        return carry

    lax.fori_loop(0, CH, _zrow, 0)
    for z in range(ZR // CH):
        pltpu.sync_copy(rows0, acc.at[pl.ds(s * ZR + z * CH, CH)])
    plsc.subcore_barrier()

    def _start(b, j):
        pltpu.async_copy(xk_h.at[gidx.at[j]], rows[b], sems[b])

    def _wait(b):
        pltpu.make_async_copy(xk_h.at[gidx.at[0]], rows[b], sems[b]).wait()

    def _scat(b, j):
        pltpu.sync_copy(rows[b], acc.at[sidx.at[j]], add=True)

    for c0 in (0, NPH):
        pltpu.sync_copy(gidx_h.at[wid, pl.ds(c0, NPH)], gidx)
        pltpu.sync_copy(sidx_h.at[wid, pl.ds(c0, NPH)], sidx)
        for b in range(NBUF):
            _start(b, b)

        def _step(t, carry):
            base = t * NBUF
            for b in range(NBUF):
                j = base + b
                _wait(b)
                _scat(b, j)
                _start(b, j + NBUF)
            return carry

        lax.fori_loop(0, (NPH - NBUF) // NBUF, _step, 0)
        for j in range(NPH - NBUF, NPH):
            b = j % NBUF
            _wait(b)
            _scat(b, j)

    plsc.subcore_barrier()
    pltpu.sync_copy(acc.at[pl.ds(s * ZR, ZR)],
                    out_h.at[pl.ds(c * NPAD + s * ZR, ZR)])


BN1 = 1000
NB = N // BN1


def _mm1_body(x_ref, w_ref, o_ref):
    o_ref[0] = jnp.dot(x_ref[...], w_ref[0], preferred_element_type=jnp.float32)


def _einsum_xw(xin, W):
    return pl.pallas_call(
        _mm1_body,
        grid=(NB, K),
        in_specs=[pl.BlockSpec((BN1, C), lambda nb, k: (nb, 0)),
                  pl.BlockSpec((1, C, C), lambda nb, k: (k, 0, 0))],
        out_specs=pl.BlockSpec((1, BN1, C), lambda nb, k: (k, nb, 0)),
        out_shape=jax.ShapeDtypeStruct((K, N, C), jnp.float32),
    )(xin, W)


_pblk0 = pl.BlockSpec((1, BN1, C), lambda i, *_: (0, i, 0))
_pblk1 = pl.BlockSpec((1, BN1, C), lambda i, *_: (1, i, 0))


def _stats_body(pa_ref, pb_ref, s_ref, q_ref):
    y = pa_ref[0] + pb_ref[0]

    @pl.when(pl.program_id(0) == 0)
    def _():
        s_ref[...] = jnp.zeros_like(s_ref)
        q_ref[...] = jnp.zeros_like(q_ref)

    s_ref[...] += jnp.sum(y, axis=0, keepdims=True)
    q_ref[...] += jnp.sum(y * y, axis=0, keepdims=True)


def _stats(p):
    return pl.pallas_call(
        _stats_body,
        grid=(NB,),
        in_specs=[_pblk0, _pblk1],
        out_specs=[pl.BlockSpec((1, C), lambda i: (0, 0)),
                   pl.BlockSpec((1, C), lambda i: (0, 0))],
        out_shape=[jax.ShapeDtypeStruct((1, C), jnp.float32),
                   jax.ShapeDtypeStruct((1, C), jnp.float32)],
    )(p, p)


def _mm2_body(pa_ref, pb_ref, s_ref, q_ref, g_ref, b_ref, w_ref, o_ref):
    mu = s_ref[0] * (1.0 / N)
    var = q_ref[0] * (1.0 / N) - mu * mu
    inv = lax.rsqrt(var + EPS) * g_ref[0]
    yn = jnp.maximum((pa_ref[0] + pb_ref[0] - mu) * inv + b_ref[0], 0.0)
    o_ref[0] = jnp.dot(yn, w_ref[0], preferred_element_type=jnp.float32)


def _einsum_bn_relu(p, ssum, sq, gamma, beta, W):
    vec = pl.BlockSpec((1, C), lambda nb, k: (0, 0))
    return pl.pallas_call(
        _mm2_body,
        grid=(NB, K),
        in_specs=[_pblk0, _pblk1, vec, vec, vec, vec,
                  pl.BlockSpec((1, C, C), lambda nb, k: (k, 0, 0))],
        out_specs=pl.BlockSpec((1, BN1, C), lambda nb, k: (k, nb, 0)),
        out_shape=jax.ShapeDtypeStruct((K, N, C), jnp.float32),
    )(p, p, ssum, sq, gamma, beta, W)


def _fin_body(pa_ref, pb_ref, s_ref, q_ref, g_ref, b_ref, x_ref, o_ref):
    mu = s_ref[0] * (1.0 / N)
    var = q_ref[0] * (1.0 / N) - mu * mu
    inv = lax.rsqrt(var + EPS) * g_ref[0]
    o_ref[...] = jnp.maximum(
        (pa_ref[0] + pb_ref[0] - mu) * inv + b_ref[0] + x_ref[...], 0.0)


def _final(p, ssum, sq, gamma, beta, x):
    blk = pl.BlockSpec((BN1, C), lambda i: (i, 0))
    vec = pl.BlockSpec((1, C), lambda i: (0, 0))
    return pl.pallas_call(
        _fin_body,
        grid=(NB,),
        in_specs=[_pblk0, _pblk1, vec, vec, vec, vec, blk],
        out_specs=blk,
        out_shape=jax.ShapeDtypeStruct((N, C), jnp.float32),
    )(p, p, ssum, sq, gamma, beta, x)


def kernel(x, edge_index, kernel_offset, W1, gamma1, beta1, W2, gamma2, beta2):
    src = edge_index[0]
    dst = edge_index[1]
    off = kernel_offset
    g1 = gamma1.reshape(1, C)
    b1 = beta1.reshape(1, C)
    g2 = gamma2.reshape(1, C)
    b2 = beta2.reshape(1, C)

    gflat = (off * N + src).reshape(NC * NS, EW)
    sflat = dst.reshape(NC * NS, EW)
    gidx = jnp.pad(gflat, ((0, 0), (0, EWP - EW))).reshape(NC * NS, NCH, CH)
    sidx = jnp.pad(sflat, ((0, 0), (0, EWP - EW)),
                   constant_values=DUMMY).reshape(NC * NS, NCH, CH)
    gidx_r = jax.new_ref(gidx, memory_space=pltpu.MemorySpace.HBM)
    sidx_r = jax.new_ref(sidx, memory_space=pltpu.MemorySpace.HBM)

    xk1 = _einsum_xw(x, W1).reshape(K * N, C)
    xk1_r = jax.new_ref(xk1, memory_space=pltpu.MemorySpace.HBM)
    p1 = _sc_gather_segsum(gidx_r, sidx_r, xk1_r).reshape(NC, NPAD, C)
    s1, q1 = _stats(p1)
    xk2 = _einsum_bn_relu(p1, s1, q1, g1, b1, W2).reshape(K * N, C)
    xk2_r = jax.new_ref(xk2, memory_space=pltpu.MemorySpace.HBM)
    p2 = _sc_gather_segsum(gidx_r, sidx_r, xk2_r).reshape(NC, NPAD, C)
    s2, q2 = _stats(p2)
    return _final(p2, s2, q2, g2, b2, x)
